# baseline (speedup 1.0000x reference)
import functools

import jax
import jax.numpy as jnp
from jax import lax
from jax.experimental import pallas as pl
from jax.experimental.pallas import tpu as pltpu
from jax.experimental.pallas import tpu_sc as plsc

_N_SAMPLE = 8
_KS0 = 0
_KS1 = 42
_KS2 = _KS0 ^ _KS1 ^ 0x1BD11BDA
_ROT_A = (13, 15, 26, 6)
_ROT_B = (17, 29, 16, 24)
_TINY = 1.1754943508222875e-38
_NEG_HUGE = -3.4e38
_IMAX = 0x7FFFFFFF


def _rotl(x, d):
    return (x << d) | lax.shift_right_logical(x, 32 - d)


def _threefry_xor(x1):
    x0 = jnp.zeros_like(x1) + _KS0
    x1 = x1 + _KS1
    sched = (
        (_ROT_B, _KS1, _KS2 + 1),
        (_ROT_A, _KS2, _KS0 + 2),
        (_ROT_B, _KS0, _KS1 + 3),
        (_ROT_A, _KS1, _KS2 + 4),
        (_ROT_B, _KS2, _KS0 + 5),
    )
    rots = _ROT_A
    for rot_next, k0, k1 in sched:
        for r in rots:
            x0 = x0 + x1
            x1 = _rotl(x1, r)
            x1 = x1 ^ x0
        x0 = x0 + k0
        x1 = x1 + k1
        rots = rot_next
    return x0 ^ x1


def _gumbel_from_bits(bits):
    fb = lax.shift_right_logical(bits, 9) | 0x3F800000
    f = lax.bitcast_convert_type(fb, jnp.float32) - jnp.float32(1.0)
    u = f + jnp.float32(_TINY)
    return -jnp.log(-jnp.log(u))


def _sc_bits(z, *, B, H, h0, nchb, CH):
    mesh = plsc.VectorSubcoreMesh(core_axis_name="c", subcore_axis_name="s")
    rows = _N_SAMPLE * B
    rows_per_w = 8
    ngroups = rows // rows_per_w

    @functools.partial(
        pl.kernel,
        out_type=jax.ShapeDtypeStruct((nchb, ngroups, rows_per_w, CH), jnp.int32),
        mesh=mesh,
        scratch_types=[pltpu.VMEM((rows_per_w, CH), jnp.int32)],
    )
    def k(z_hbm, out_hbm, buf):
        del z_hbm
        cid = lax.axis_index("c")
        sid = lax.axis_index("s")
        wid = sid * 2 + cid

        nsub = max(1, CH // 512)

        def chunk_body(c, _):
            @plsc.parallel_loop(0, rows_per_w * nsub)
            def sub_body(t):
                p = t // nsub
                u = t - p * nsub
                q = wid * rows_per_w + p
                s = q // B
                b = q - s * B
                base = s * (B * H) + b * H + h0 + c * CH + u * 512
                for v in range(min(CH, 512) // 16):
                    x1 = base + v * 16 + lax.broadcasted_iota(
                        jnp.int32, (16,), 0
                    )
                    buf[p, pl.ds(u * 512 + v * 16, 16)] = _threefry_xor(x1)

            pltpu.sync_copy(buf, out_hbm.at[c, wid])
            return 0

        lax.fori_loop(0, nchb, chunk_body, 0)

    return k(z)


def _sample_a_body(z_ref, idx_ref, val_ref, acc_val, acc_idx, *, nchunks, B, H, CH):
    c = pl.program_id(0)

    @pl.when(c == 0)
    def _init():
        acc_val[...] = jnp.full((_N_SAMPLE, B, CH), _NEG_HUGE, jnp.float32)
        acc_idx[...] = jnp.full((_N_SAMPLE, B, CH), _IMAX, jnp.int32)

    z = z_ref[...]
    h = c * CH + lax.broadcasted_iota(jnp.int32, (B, CH), 1)
    base = lax.broadcasted_iota(jnp.int32, (B, CH), 0) * H + h
    for s in range(_N_SAMPLE):
        bits = _threefry_xor(base + s * (B * H))
        score = _gumbel_from_bits(bits) + z
        take = score > acc_val[s]
        acc_val[s] = jnp.where(take, score, acc_val[s])
        acc_idx[s] = jnp.where(take, h, acc_idx[s])

    @pl.when(c == nchunks - 1)
    def _finalize():
        for s in range(_N_SAMPLE):
            av = acc_val[s]
            mx = jnp.max(av, axis=1, keepdims=True)
            sel = jnp.where(av == mx, acc_idx[s], _IMAX)
            idx_ref[:, s : s + 1] = jnp.min(sel, axis=1, keepdims=True)
            val_ref[:, s : s + 1] = mx


def _sample_b_body(
    bits_ref, z_ref, aval_ref, aidx_ref, out_ref, acc_val, acc_idx, *,
    nchb, B, H, h0, CH,
):
    c = pl.program_id(0)
    G = B // 8

    @pl.when(c == 0)
    def _init():
        acc_val[...] = jnp.full((_N_SAMPLE, G, 8, CH), _NEG_HUGE, jnp.float32)
        acc_idx[...] = jnp.full((_N_SAMPLE, G, 8, CH), _IMAX, jnp.int32)

    z = z_ref[...].reshape(G, 8, CH)
    h = h0 + c * CH + lax.broadcasted_iota(jnp.int32, (G, 8, CH), 2)
    valid = h < H
    for s in range(_N_SAMPLE):
        bits = bits_ref[0, s * G : (s + 1) * G]
        score = _gumbel_from_bits(bits) + z
        score = jnp.where(valid, score, _NEG_HUGE)
        take = score > acc_val[s]
        acc_val[s] = jnp.where(take, score, acc_val[s])
        acc_idx[s] = jnp.where(take, h, acc_idx[s])

    @pl.when(c == nchb - 1)
    def _finalize():
        va = aval_ref[...]
        ia = aidx_ref[...]
        for s in range(_N_SAMPLE):
            av = acc_val[s]
            mx = jnp.max(av, axis=2, keepdims=True)
            sel = jnp.where(av == mx, acc_idx[s], _IMAX)
            mi = jnp.min(sel, axis=2, keepdims=True).reshape(B, 1)
            better = mx.reshape(B, 1) > va[:, s : s + 1]
            out_ref[:, s : s + 1] = jnp.where(better, mi, ia[:, s : s + 1])


def _onehot_body(samp_ref, out_ref, *, B, BH):
    c = pl.program_id(0)
    hidx = c * BH + lax.broadcasted_iota(jnp.int32, (_N_SAMPLE, B, BH), 2)
    samp = samp_ref[...][:, :, None]
    out_ref[...] = jnp.where(hidx == samp, jnp.float32(1.0), jnp.float32(0.0))


def kernel(z):
    B, H = z.shape
    CH = 512
    CHB = 1024
    h0 = (H * 18 // 25) // CHB * CHB
    hs = H - h0
    nchb = pl.cdiv(hs, CHB)
    bits = _sc_bits(z, B=B, H=H, h0=h0, nchb=nchb, CH=CHB)

    ncha = h0 // CH
    idx_a, val_a = pl.pallas_call(
        functools.partial(_sample_a_body, nchunks=ncha, B=B, H=H, CH=CH),
        grid=(ncha,),
        in_specs=[pl.BlockSpec((B, CH), lambda c: (0, c))],
        out_specs=[
            pl.BlockSpec((B, _N_SAMPLE), lambda c: (0, 0)),
            pl.BlockSpec((B, _N_SAMPLE), lambda c: (0, 0)),
        ],
        out_shape=[
            jax.ShapeDtypeStruct((B, _N_SAMPLE), jnp.int32),
            jax.ShapeDtypeStruct((B, _N_SAMPLE), jnp.float32),
        ],
        scratch_shapes=[
            pltpu.VMEM((_N_SAMPLE, B, CH), jnp.float32),
            pltpu.VMEM((_N_SAMPLE, B, CH), jnp.int32),
        ],
    )(z)

    samples_bn = pl.pallas_call(
        functools.partial(_sample_b_body, nchb=nchb, B=B, H=H, h0=h0, CH=CHB),
        grid=(nchb,),
        in_specs=[
            pl.BlockSpec((1, _N_SAMPLE * B // 8, 8, CHB), lambda c: (c, 0, 0, 0)),
            pl.BlockSpec((B, CHB), lambda c, _h0=h0 // CHB: (0, c + _h0)),
            pl.BlockSpec((B, _N_SAMPLE), lambda c: (0, 0)),
            pl.BlockSpec((B, _N_SAMPLE), lambda c: (0, 0)),
        ],
        out_specs=pl.BlockSpec((B, _N_SAMPLE), lambda c: (0, 0)),
        out_shape=jax.ShapeDtypeStruct((B, _N_SAMPLE), jnp.int32),
        scratch_shapes=[
            pltpu.VMEM((_N_SAMPLE, B // 8, 8, CHB), jnp.float32),
            pltpu.VMEM((_N_SAMPLE, B // 8, 8, CHB), jnp.int32),
        ],
    )(bits, z, val_a, idx_a)

    samp = samples_bn.reshape(_N_SAMPLE, B)
    BH = 2048
    nbh = pl.cdiv(H, BH)
    return pl.pallas_call(
        functools.partial(_onehot_body, B=B, BH=BH),
        grid=(nbh,),
        in_specs=[pl.BlockSpec((_N_SAMPLE, B), lambda c: (0, 0))],
        out_specs=pl.BlockSpec((_N_SAMPLE, B, BH), lambda c: (0, 0, c)),
        out_shape=jax.ShapeDtypeStruct((_N_SAMPLE, B, H), jnp.float32),
    )(samp)


# speedup vs baseline: 1.0340x; 1.0340x over previous
import functools

import jax
import jax.numpy as jnp
from jax import lax
from jax.experimental import pallas as pl
from jax.experimental.pallas import tpu as pltpu
from jax.experimental.pallas import tpu_sc as plsc

_N_SAMPLE = 8
_KS0 = 0
_KS1 = 42
_KS2 = _KS0 ^ _KS1 ^ 0x1BD11BDA
_ROT_A = (13, 15, 26, 6)
_ROT_B = (17, 29, 16, 24)
_TINY = 1.1754943508222875e-38
_NEG_HUGE = -3.4e38
_IMAX = 0x7FFFFFFF


def _rotl(x, d):
    return (x << d) | lax.shift_right_logical(x, 32 - d)


def _threefry_xor(x1):
    x0 = jnp.zeros_like(x1) + _KS0
    x1 = x1 + _KS1
    sched = (
        (_ROT_B, _KS1, _KS2 + 1),
        (_ROT_A, _KS2, _KS0 + 2),
        (_ROT_B, _KS0, _KS1 + 3),
        (_ROT_A, _KS1, _KS2 + 4),
        (_ROT_B, _KS2, _KS0 + 5),
    )
    rots = _ROT_A
    for rot_next, k0, k1 in sched:
        for r in rots:
            x0 = x0 + x1
            x1 = _rotl(x1, r)
            x1 = x1 ^ x0
        x0 = x0 + k0
        x1 = x1 + k1
        rots = rot_next
    return x0 ^ x1


def _gumbel_from_bits(bits):
    fb = lax.shift_right_logical(bits, 9) | 0x3F800000
    f = lax.bitcast_convert_type(fb, jnp.float32) - jnp.float32(1.0)
    u = f + jnp.float32(_TINY)
    return -jnp.log(-jnp.log(u))


def _sc_bits(z, *, B, H, h0, nchb, CH):
    mesh = plsc.VectorSubcoreMesh(core_axis_name="c", subcore_axis_name="s")
    rows = _N_SAMPLE * B
    rows_per_w = 8
    ngroups = rows // rows_per_w

    @functools.partial(
        pl.kernel,
        out_type=jax.ShapeDtypeStruct((nchb, ngroups, rows_per_w, CH), jnp.int32),
        mesh=mesh,
        scratch_types=[pltpu.VMEM((rows_per_w, CH), jnp.int32)],
    )
    def k(z_hbm, out_hbm, buf):
        del z_hbm
        cid = lax.axis_index("c")
        sid = lax.axis_index("s")
        wid = sid * 2 + cid

        nsub = max(1, CH // 512)

        def chunk_body(c, _):
            @plsc.parallel_loop(0, rows_per_w * nsub)
            def sub_body(t):
                p = t // nsub
                u = t - p * nsub
                q = wid * rows_per_w + p
                s = q // B
                b = q - s * B
                base = s * (B * H) + b * H + h0 + c * CH + u * 512
                for v in range(min(CH, 512) // 16):
                    x1 = base + v * 16 + lax.broadcasted_iota(
                        jnp.int32, (16,), 0
                    )
                    buf[p, pl.ds(u * 512 + v * 16, 16)] = _threefry_xor(x1)

            pltpu.sync_copy(buf, out_hbm.at[c, wid])
            return 0

        lax.fori_loop(0, nchb, chunk_body, 0)

    return k(z)


def _sample_a_body(z_ref, idx_ref, val_ref, acc_val, acc_idx, *, nchunks, B, H, CH):
    c = pl.program_id(0)

    @pl.when(c == 0)
    def _init():
        acc_val[...] = jnp.full((_N_SAMPLE, B, CH), _NEG_HUGE, jnp.float32)
        acc_idx[...] = jnp.full((_N_SAMPLE, B, CH), _IMAX, jnp.int32)

    z = z_ref[...]
    h = c * CH + lax.broadcasted_iota(jnp.int32, (B, CH), 1)
    base = lax.broadcasted_iota(jnp.int32, (B, CH), 0) * H + h
    for s in range(_N_SAMPLE):
        bits = _threefry_xor(base + s * (B * H))
        score = _gumbel_from_bits(bits) + z
        take = score > acc_val[s]
        acc_val[s] = jnp.where(take, score, acc_val[s])
        acc_idx[s] = jnp.where(take, h, acc_idx[s])

    @pl.when(c == nchunks - 1)
    def _finalize():
        for s in range(_N_SAMPLE):
            av = acc_val[s]
            mx = jnp.max(av, axis=1, keepdims=True)
            sel = jnp.where(av == mx, acc_idx[s], _IMAX)
            idx_ref[:, s : s + 1] = jnp.min(sel, axis=1, keepdims=True)
            val_ref[:, s : s + 1] = mx


def _sample_b_body(
    bits_ref, z_ref, aval_ref, aidx_ref, out_ref, acc_val, acc_idx, *,
    nchb, B, H, h0, CH,
):
    c = pl.program_id(0)
    G = B // 8

    @pl.when(c == 0)
    def _init():
        acc_val[...] = jnp.full((_N_SAMPLE, G, 8, CH), _NEG_HUGE, jnp.float32)
        acc_idx[...] = jnp.full((_N_SAMPLE, G, 8, CH), _IMAX, jnp.int32)

    z = z_ref[...].reshape(G, 8, CH)
    h = h0 + c * CH + lax.broadcasted_iota(jnp.int32, (G, 8, CH), 2)
    valid = h < H
    for s in range(_N_SAMPLE):
        bits = bits_ref[0, s * G : (s + 1) * G]
        score = _gumbel_from_bits(bits) + z
        score = jnp.where(valid, score, _NEG_HUGE)
        take = score > acc_val[s]
        acc_val[s] = jnp.where(take, score, acc_val[s])
        acc_idx[s] = jnp.where(take, h, acc_idx[s])

    @pl.when(c == nchb - 1)
    def _finalize():
        va = aval_ref[...]
        ia = aidx_ref[...]
        for s in range(_N_SAMPLE):
            av = acc_val[s]
            mx = jnp.max(av, axis=2, keepdims=True)
            sel = jnp.where(av == mx, acc_idx[s], _IMAX)
            mi = jnp.min(sel, axis=2, keepdims=True).reshape(B, 1)
            better = mx.reshape(B, 1) > va[:, s : s + 1]
            out_ref[:, s : s + 1] = jnp.where(better, mi, ia[:, s : s + 1])


def _onehot_body(samp_ref, out_ref, *, B, BH):
    c = pl.program_id(0)
    hidx = c * BH + lax.broadcasted_iota(jnp.int32, (_N_SAMPLE, B, BH), 2)
    samp = samp_ref[...][:, :, None]
    out_ref[...] = jnp.where(hidx == samp, jnp.float32(1.0), jnp.float32(0.0))


def kernel(z):
    B, H = z.shape
    CH = 512
    CHB = 2048
    h0 = (H * 18 // 25) // CHB * CHB
    hs = H - h0
    nchb = pl.cdiv(hs, CHB)
    bits = _sc_bits(z, B=B, H=H, h0=h0, nchb=nchb, CH=CHB)

    ncha = h0 // CH
    idx_a, val_a = pl.pallas_call(
        functools.partial(_sample_a_body, nchunks=ncha, B=B, H=H, CH=CH),
        grid=(ncha,),
        in_specs=[pl.BlockSpec((B, CH), lambda c: (0, c))],
        out_specs=[
            pl.BlockSpec((B, _N_SAMPLE), lambda c: (0, 0)),
            pl.BlockSpec((B, _N_SAMPLE), lambda c: (0, 0)),
        ],
        out_shape=[
            jax.ShapeDtypeStruct((B, _N_SAMPLE), jnp.int32),
            jax.ShapeDtypeStruct((B, _N_SAMPLE), jnp.float32),
        ],
        scratch_shapes=[
            pltpu.VMEM((_N_SAMPLE, B, CH), jnp.float32),
            pltpu.VMEM((_N_SAMPLE, B, CH), jnp.int32),
        ],
    )(z)

    samples_bn = pl.pallas_call(
        functools.partial(_sample_b_body, nchb=nchb, B=B, H=H, h0=h0, CH=CHB),
        grid=(nchb,),
        in_specs=[
            pl.BlockSpec((1, _N_SAMPLE * B // 8, 8, CHB), lambda c: (c, 0, 0, 0)),
            pl.BlockSpec((B, CHB), lambda c, _h0=h0 // CHB: (0, c + _h0)),
            pl.BlockSpec((B, _N_SAMPLE), lambda c: (0, 0)),
            pl.BlockSpec((B, _N_SAMPLE), lambda c: (0, 0)),
        ],
        out_specs=pl.BlockSpec((B, _N_SAMPLE), lambda c: (0, 0)),
        out_shape=jax.ShapeDtypeStruct((B, _N_SAMPLE), jnp.int32),
        scratch_shapes=[
            pltpu.VMEM((_N_SAMPLE, B // 8, 8, CHB), jnp.float32),
            pltpu.VMEM((_N_SAMPLE, B // 8, 8, CHB), jnp.int32),
        ],
    )(bits, z, val_a, idx_a)

    samp = samples_bn.reshape(_N_SAMPLE, B)
    BH = 4096
    nbh = pl.cdiv(H, BH)
    return pl.pallas_call(
        functools.partial(_onehot_body, B=B, BH=BH),
        grid=(nbh,),
        in_specs=[pl.BlockSpec((_N_SAMPLE, B), lambda c: (0, 0))],
        out_specs=pl.BlockSpec((_N_SAMPLE, B, BH), lambda c: (0, 0, c)),
        out_shape=jax.ShapeDtypeStruct((_N_SAMPLE, B, H), jnp.float32),
    )(samp)


# speedup vs baseline: 1.0344x; 1.0004x over previous
import functools

import jax
import jax.numpy as jnp
from jax import lax
from jax.experimental import pallas as pl
from jax.experimental.pallas import tpu as pltpu
from jax.experimental.pallas import tpu_sc as plsc

_N_SAMPLE = 8
_KS0 = 0
_KS1 = 42
_KS2 = _KS0 ^ _KS1 ^ 0x1BD11BDA
_ROT_A = (13, 15, 26, 6)
_ROT_B = (17, 29, 16, 24)
_TINY = 1.1754943508222875e-38
_NEG_HUGE = -3.4e38
_IMAX = 0x7FFFFFFF


def _rotl(x, d):
    return (x << d) | lax.shift_right_logical(x, 32 - d)


def _threefry_xor(x1):
    x0 = jnp.zeros_like(x1) + _KS0
    x1 = x1 + _KS1
    sched = (
        (_ROT_B, _KS1, _KS2 + 1),
        (_ROT_A, _KS2, _KS0 + 2),
        (_ROT_B, _KS0, _KS1 + 3),
        (_ROT_A, _KS1, _KS2 + 4),
        (_ROT_B, _KS2, _KS0 + 5),
    )
    rots = _ROT_A
    for rot_next, k0, k1 in sched:
        for r in rots:
            x0 = x0 + x1
            x1 = _rotl(x1, r)
            x1 = x1 ^ x0
        x0 = x0 + k0
        x1 = x1 + k1
        rots = rot_next
    return x0 ^ x1


def _gumbel_from_bits(bits):
    fb = lax.shift_right_logical(bits, 9) | 0x3F800000
    f = lax.bitcast_convert_type(fb, jnp.float32) - jnp.float32(1.0)
    u = f + jnp.float32(_TINY)
    return -jnp.log(-jnp.log(u))


def _sc_bits(z, *, B, H, h0, nchb, CH):
    mesh = plsc.VectorSubcoreMesh(core_axis_name="c", subcore_axis_name="s")
    rows = _N_SAMPLE * B
    rows_per_w = 8
    ngroups = rows // rows_per_w

    @functools.partial(
        pl.kernel,
        out_type=jax.ShapeDtypeStruct((nchb, ngroups, rows_per_w, CH), jnp.int32),
        mesh=mesh,
        scratch_types=[pltpu.VMEM((rows_per_w, CH), jnp.int32)],
    )
    def k(z_hbm, out_hbm, buf):
        del z_hbm
        cid = lax.axis_index("c")
        sid = lax.axis_index("s")
        wid = sid * 2 + cid

        nsub = max(1, CH // 512)

        def chunk_body(c, _):
            @plsc.parallel_loop(0, rows_per_w * nsub)
            def sub_body(t):
                p = t // nsub
                u = t - p * nsub
                q = wid * rows_per_w + p
                s = q // B
                b = q - s * B
                base = s * (B * H) + b * H + h0 + c * CH + u * 512
                for v in range(min(CH, 512) // 16):
                    x1 = base + v * 16 + lax.broadcasted_iota(
                        jnp.int32, (16,), 0
                    )
                    buf[p, pl.ds(u * 512 + v * 16, 16)] = _threefry_xor(x1)

            pltpu.sync_copy(buf, out_hbm.at[c, wid])
            return 0

        lax.fori_loop(0, nchb, chunk_body, 0)

    return k(z)


def _sample_a_body(z_ref, idx_ref, val_ref, acc_val, acc_idx, *, nchunks, B, H, CH):
    c = pl.program_id(0)

    @pl.when(c == 0)
    def _init():
        acc_val[...] = jnp.full((_N_SAMPLE, B, CH), _NEG_HUGE, jnp.float32)
        acc_idx[...] = jnp.full((_N_SAMPLE, B, CH), _IMAX, jnp.int32)

    z = z_ref[...]
    h = c * CH + lax.broadcasted_iota(jnp.int32, (B, CH), 1)
    base = lax.broadcasted_iota(jnp.int32, (B, CH), 0) * H + h
    for s in range(_N_SAMPLE):
        bits = _threefry_xor(base + s * (B * H))
        score = _gumbel_from_bits(bits) + z
        take = score > acc_val[s]
        acc_val[s] = jnp.where(take, score, acc_val[s])
        acc_idx[s] = jnp.where(take, h, acc_idx[s])

    @pl.when(c == nchunks - 1)
    def _finalize():
        for s in range(_N_SAMPLE):
            av = acc_val[s]
            mx = jnp.max(av, axis=1, keepdims=True)
            sel = jnp.where(av == mx, acc_idx[s], _IMAX)
            idx_ref[:, s : s + 1] = jnp.min(sel, axis=1, keepdims=True)
            val_ref[:, s : s + 1] = mx


def _sample_b_body(
    bits_ref, z_ref, aval_ref, aidx_ref, out_ref, acc_val, acc_idx, *,
    nchb, B, H, h0, CH,
):
    c = pl.program_id(0)
    G = B // 8

    @pl.when(c == 0)
    def _init():
        acc_val[...] = jnp.full((_N_SAMPLE, G, 8, CH), _NEG_HUGE, jnp.float32)
        acc_idx[...] = jnp.full((_N_SAMPLE, G, 8, CH), _IMAX, jnp.int32)

    z = z_ref[...].reshape(G, 8, CH)
    h = h0 + c * CH + lax.broadcasted_iota(jnp.int32, (G, 8, CH), 2)
    valid = h < H
    for s in range(_N_SAMPLE):
        bits = bits_ref[0, s * G : (s + 1) * G]
        score = _gumbel_from_bits(bits) + z
        score = jnp.where(valid, score, _NEG_HUGE)
        take = score > acc_val[s]
        acc_val[s] = jnp.where(take, score, acc_val[s])
        acc_idx[s] = jnp.where(take, h, acc_idx[s])

    @pl.when(c == nchb - 1)
    def _finalize():
        va = aval_ref[...]
        ia = aidx_ref[...]
        for s in range(_N_SAMPLE):
            av = acc_val[s]
            mx = jnp.max(av, axis=2, keepdims=True)
            sel = jnp.where(av == mx, acc_idx[s], _IMAX)
            mi = jnp.min(sel, axis=2, keepdims=True).reshape(B, 1)
            better = mx.reshape(B, 1) > va[:, s : s + 1]
            out_ref[:, s : s + 1] = jnp.where(better, mi, ia[:, s : s + 1])


def _onehot_body(samp_ref, out_ref, *, B, BH):
    c = pl.program_id(0)
    hidx = c * BH + lax.broadcasted_iota(jnp.int32, (_N_SAMPLE, B, BH), 2)
    samp = samp_ref[...][:, :, None]
    out_ref[...] = jnp.where(hidx == samp, jnp.float32(1.0), jnp.float32(0.0))


def kernel(z):
    B, H = z.shape
    CH = 1024
    CHB = 2048
    h0 = (H * 18 // 25) // CHB * CHB
    hs = H - h0
    nchb = pl.cdiv(hs, CHB)
    bits = _sc_bits(z, B=B, H=H, h0=h0, nchb=nchb, CH=CHB)

    ncha = h0 // CH
    idx_a, val_a = pl.pallas_call(
        functools.partial(_sample_a_body, nchunks=ncha, B=B, H=H, CH=CH),
        grid=(ncha,),
        in_specs=[pl.BlockSpec((B, CH), lambda c: (0, c))],
        out_specs=[
            pl.BlockSpec((B, _N_SAMPLE), lambda c: (0, 0)),
            pl.BlockSpec((B, _N_SAMPLE), lambda c: (0, 0)),
        ],
        out_shape=[
            jax.ShapeDtypeStruct((B, _N_SAMPLE), jnp.int32),
            jax.ShapeDtypeStruct((B, _N_SAMPLE), jnp.float32),
        ],
        scratch_shapes=[
            pltpu.VMEM((_N_SAMPLE, B, CH), jnp.float32),
            pltpu.VMEM((_N_SAMPLE, B, CH), jnp.int32),
        ],
    )(z)

    samples_bn = pl.pallas_call(
        functools.partial(_sample_b_body, nchb=nchb, B=B, H=H, h0=h0, CH=CHB),
        grid=(nchb,),
        in_specs=[
            pl.BlockSpec((1, _N_SAMPLE * B // 8, 8, CHB), lambda c: (c, 0, 0, 0)),
            pl.BlockSpec((B, CHB), lambda c, _h0=h0 // CHB: (0, c + _h0)),
            pl.BlockSpec((B, _N_SAMPLE), lambda c: (0, 0)),
            pl.BlockSpec((B, _N_SAMPLE), lambda c: (0, 0)),
        ],
        out_specs=pl.BlockSpec((B, _N_SAMPLE), lambda c: (0, 0)),
        out_shape=jax.ShapeDtypeStruct((B, _N_SAMPLE), jnp.int32),
        scratch_shapes=[
            pltpu.VMEM((_N_SAMPLE, B // 8, 8, CHB), jnp.float32),
            pltpu.VMEM((_N_SAMPLE, B // 8, 8, CHB), jnp.int32),
        ],
    )(bits, z, val_a, idx_a)

    samp = samples_bn.reshape(_N_SAMPLE, B)
    BH = 4096
    nbh = pl.cdiv(H, BH)
    return pl.pallas_call(
        functools.partial(_onehot_body, B=B, BH=BH),
        grid=(nbh,),
        in_specs=[pl.BlockSpec((_N_SAMPLE, B), lambda c: (0, 0))],
        out_specs=pl.BlockSpec((_N_SAMPLE, B, BH), lambda c: (0, 0, c)),
        out_shape=jax.ShapeDtypeStruct((_N_SAMPLE, B, H), jnp.float32),
    )(samp)


# speedup vs baseline: 1.0505x; 1.0155x over previous
import functools

import jax
import jax.numpy as jnp
from jax import lax
from jax.experimental import pallas as pl
from jax.experimental.pallas import tpu as pltpu
from jax.experimental.pallas import tpu_sc as plsc

_N_SAMPLE = 8
_KS0 = 0
_KS1 = 42
_KS2 = _KS0 ^ _KS1 ^ 0x1BD11BDA
_ROT_A = (13, 15, 26, 6)
_ROT_B = (17, 29, 16, 24)
_TINY = 1.1754943508222875e-38
_NEG_HUGE = -3.4e38
_IMAX = 0x7FFFFFFF


def _rotl(x, d):
    return (x << d) | lax.shift_right_logical(x, 32 - d)


def _threefry_xor(x1):
    x0 = jnp.zeros_like(x1) + _KS0
    x1 = x1 + _KS1
    sched = (
        (_ROT_B, _KS1, _KS2 + 1),
        (_ROT_A, _KS2, _KS0 + 2),
        (_ROT_B, _KS0, _KS1 + 3),
        (_ROT_A, _KS1, _KS2 + 4),
        (_ROT_B, _KS2, _KS0 + 5),
    )
    rots = _ROT_A
    for rot_next, k0, k1 in sched:
        for r in rots:
            x0 = x0 + x1
            x1 = _rotl(x1, r)
            x1 = x1 ^ x0
        x0 = x0 + k0
        x1 = x1 + k1
        rots = rot_next
    return x0 ^ x1


def _gumbel_from_bits(bits):
    fb = lax.shift_right_logical(bits, 9) | 0x3F800000
    f = lax.bitcast_convert_type(fb, jnp.float32) - jnp.float32(1.0)
    u = f + jnp.float32(_TINY)
    return -jnp.log(-jnp.log(u))


def _sc_bits(z, *, B, H, h0, nchb, CH):
    mesh = plsc.VectorSubcoreMesh(core_axis_name="c", subcore_axis_name="s")
    rows = _N_SAMPLE * B
    rows_per_w = 8
    ngroups = rows // rows_per_w

    @functools.partial(
        pl.kernel,
        out_type=jax.ShapeDtypeStruct((nchb, ngroups, rows_per_w, CH), jnp.int32),
        mesh=mesh,
        scratch_types=[pltpu.VMEM((rows_per_w, CH), jnp.int32)],
    )
    def k(z_hbm, out_hbm, buf):
        del z_hbm
        cid = lax.axis_index("c")
        sid = lax.axis_index("s")
        wid = sid * 2 + cid

        nsub = max(1, CH // 512)

        def chunk_body(c, _):
            @plsc.parallel_loop(0, rows_per_w * nsub)
            def sub_body(t):
                p = t // nsub
                u = t - p * nsub
                q = wid * rows_per_w + p
                s = q // B
                b = q - s * B
                base = s * (B * H) + b * H + h0 + c * CH + u * 512
                for v in range(min(CH, 512) // 16):
                    x1 = base + v * 16 + lax.broadcasted_iota(
                        jnp.int32, (16,), 0
                    )
                    buf[p, pl.ds(u * 512 + v * 16, 16)] = _threefry_xor(x1)

            pltpu.sync_copy(buf, out_hbm.at[c, wid])
            return 0

        lax.fori_loop(0, nchb, chunk_body, 0)

    return k(z)


def _sample_a_body(z_ref, idx_ref, val_ref, acc_val, acc_idx, *, nchunks, B, H, CH):
    c = pl.program_id(0)

    @pl.when(c == 0)
    def _init():
        acc_val[...] = jnp.full((_N_SAMPLE, B, CH), _NEG_HUGE, jnp.float32)
        acc_idx[...] = jnp.full((_N_SAMPLE, B, CH), _IMAX, jnp.int32)

    z = z_ref[...]
    h = c * CH + lax.broadcasted_iota(jnp.int32, (B, CH), 1)
    base = lax.broadcasted_iota(jnp.int32, (B, CH), 0) * H + h
    for s in range(_N_SAMPLE):
        bits = _threefry_xor(base + s * (B * H))
        score = _gumbel_from_bits(bits) + z
        take = score > acc_val[s]
        acc_val[s] = jnp.where(take, score, acc_val[s])
        acc_idx[s] = jnp.where(take, h, acc_idx[s])

    @pl.when(c == nchunks - 1)
    def _finalize():
        for s in range(_N_SAMPLE):
            av = acc_val[s]
            mx = jnp.max(av, axis=1, keepdims=True)
            sel = jnp.where(av == mx, acc_idx[s], _IMAX)
            idx_ref[:, s : s + 1] = jnp.min(sel, axis=1, keepdims=True)
            val_ref[:, s : s + 1] = mx


def _sample_b_body(
    bits_ref, z_ref, aval_ref, aidx_ref, out_ref, acc_val, acc_idx, *,
    nchb, B, H, h0, CH,
):
    c = pl.program_id(0)
    G = B // 8

    @pl.when(c == 0)
    def _init():
        acc_val[...] = jnp.full((_N_SAMPLE, G, 8, CH), _NEG_HUGE, jnp.float32)
        acc_idx[...] = jnp.full((_N_SAMPLE, G, 8, CH), _IMAX, jnp.int32)

    z = z_ref[...].reshape(G, 8, CH)
    h = h0 + c * CH + lax.broadcasted_iota(jnp.int32, (G, 8, CH), 2)
    valid = h < H
    for s in range(_N_SAMPLE):
        bits = bits_ref[0, s * G : (s + 1) * G]
        score = _gumbel_from_bits(bits) + z
        score = jnp.where(valid, score, _NEG_HUGE)
        take = score > acc_val[s]
        acc_val[s] = jnp.where(take, score, acc_val[s])
        acc_idx[s] = jnp.where(take, h, acc_idx[s])

    @pl.when(c == nchb - 1)
    def _finalize():
        va = aval_ref[...]
        ia = aidx_ref[...]
        for s in range(_N_SAMPLE):
            av = acc_val[s]
            mx = jnp.max(av, axis=2, keepdims=True)
            sel = jnp.where(av == mx, acc_idx[s], _IMAX)
            mi = jnp.min(sel, axis=2, keepdims=True).reshape(B, 1)
            better = mx.reshape(B, 1) > va[:, s : s + 1]
            out_ref[:, s : s + 1] = jnp.where(better, mi, ia[:, s : s + 1])


def _onehot_body(samp_ref, out_ref, *, B, BH):
    c = pl.program_id(0)
    hidx = c * BH + lax.broadcasted_iota(jnp.int32, (_N_SAMPLE, B, BH), 2)
    samp = samp_ref[...][:, :, None]
    out_ref[...] = jnp.where(hidx == samp, jnp.float32(1.0), jnp.float32(0.0))


def kernel(z):
    B, H = z.shape
    CH = 1024
    CHB = 1024
    h0 = 71 * CHB
    hs = H - h0
    nchb = pl.cdiv(hs, CHB)
    bits = _sc_bits(z, B=B, H=H, h0=h0, nchb=nchb, CH=CHB)

    ncha = h0 // CH
    idx_a, val_a = pl.pallas_call(
        functools.partial(_sample_a_body, nchunks=ncha, B=B, H=H, CH=CH),
        grid=(ncha,),
        in_specs=[pl.BlockSpec((B, CH), lambda c: (0, c))],
        out_specs=[
            pl.BlockSpec((B, _N_SAMPLE), lambda c: (0, 0)),
            pl.BlockSpec((B, _N_SAMPLE), lambda c: (0, 0)),
        ],
        out_shape=[
            jax.ShapeDtypeStruct((B, _N_SAMPLE), jnp.int32),
            jax.ShapeDtypeStruct((B, _N_SAMPLE), jnp.float32),
        ],
        scratch_shapes=[
            pltpu.VMEM((_N_SAMPLE, B, CH), jnp.float32),
            pltpu.VMEM((_N_SAMPLE, B, CH), jnp.int32),
        ],
    )(z)

    samples_bn = pl.pallas_call(
        functools.partial(_sample_b_body, nchb=nchb, B=B, H=H, h0=h0, CH=CHB),
        grid=(nchb,),
        in_specs=[
            pl.BlockSpec((1, _N_SAMPLE * B // 8, 8, CHB), lambda c: (c, 0, 0, 0)),
            pl.BlockSpec((B, CHB), lambda c, _h0=h0 // CHB: (0, c + _h0)),
            pl.BlockSpec((B, _N_SAMPLE), lambda c: (0, 0)),
            pl.BlockSpec((B, _N_SAMPLE), lambda c: (0, 0)),
        ],
        out_specs=pl.BlockSpec((B, _N_SAMPLE), lambda c: (0, 0)),
        out_shape=jax.ShapeDtypeStruct((B, _N_SAMPLE), jnp.int32),
        scratch_shapes=[
            pltpu.VMEM((_N_SAMPLE, B // 8, 8, CHB), jnp.float32),
            pltpu.VMEM((_N_SAMPLE, B // 8, 8, CHB), jnp.int32),
        ],
    )(bits, z, val_a, idx_a)

    samp = samples_bn.reshape(_N_SAMPLE, B)
    BH = 4096
    nbh = pl.cdiv(H, BH)
    return pl.pallas_call(
        functools.partial(_onehot_body, B=B, BH=BH),
        grid=(nbh,),
        in_specs=[pl.BlockSpec((_N_SAMPLE, B), lambda c: (0, 0))],
        out_specs=pl.BlockSpec((_N_SAMPLE, B, BH), lambda c: (0, 0, c)),
        out_shape=jax.ShapeDtypeStruct((_N_SAMPLE, B, H), jnp.float32),
    )(samp)
